# baseline (speedup 1.0000x reference)
import functools

import jax
import jax.numpy as jnp
from jax import lax
from jax.experimental import pallas as pl
from jax.experimental.pallas import tpu as pltpu, tpu_sc as plsc

_L = 16


def _sc_body(x_hbm, out_hbm, vals_v, row_v, sem, *, img_w, img_h):
    cid = lax.axis_index("c")
    sid = lax.axis_index("s")

    @pl.when((cid == 0) & (sid == 0))
    def _work():
        copies = [
            pltpu.make_async_copy(
                x_hbm.at[0, c, 0, pl.ds(0, _L)], vals_v.at[c], sem
            )
            for c in range(5)
        ]
        for cp in copies:
            cp.start()
        for cp in copies:
            cp.wait()
        lanes = lax.broadcasted_iota(jnp.int32, (_L,), 0)
        lane0 = lanes == 0
        zeros = jnp.zeros((_L,), jnp.float32)

        def lane0_scalar(c):
            return lax.reduce(
                jnp.where(lane0, vals_v[c], zeros), 0.0, lax.add, (0,)
            )

        cx = lane0_scalar(0)
        cy = lane0_scalar(1)
        dw = lane0_scalar(2) * 0.5
        dh = lane0_scalar(3) * 0.5
        score = lane0_scalar(4)
        x1 = jnp.clip(cx - dw, 0.0, img_w)
        y1 = jnp.clip(cy - dh, 0.0, img_h)
        x2 = jnp.clip(cx + dw, 0.0, img_w)
        y2 = jnp.clip(cy + dh, 0.0, img_h)
        row = zeros
        for k, v in enumerate((x1, y1, x2, y2, score)):
            row = jnp.where(lanes == k + 1, v, row)
        row_v[...] = row
        pltpu.make_async_copy(row_v.at[pl.ds(0, 7)], out_hbm.at[0], sem).start()
        pltpu.make_async_copy(row_v.at[pl.ds(0, 7)], out_hbm.at[0], sem).wait()


def kernel(x):
    img_h, img_w = float(x.shape[2]), float(x.shape[3])
    body = functools.partial(_sc_body, img_w=img_w, img_h=img_h)
    mesh = plsc.VectorSubcoreMesh(core_axis_name="c", subcore_axis_name="s")
    return pl.kernel(
        body,
        mesh=mesh,
        compiler_params=pltpu.CompilerParams(needs_layout_passes=False),
        out_type=jax.ShapeDtypeStruct((1, 7), jnp.float32),
        scratch_types=[
            pltpu.VMEM((5, _L), jnp.float32),
            pltpu.VMEM((_L,), jnp.float32),
            pltpu.SemaphoreType.DMA,
        ],
    )(x)


# speedup vs baseline: 2.9835x; 2.9835x over previous
import functools

import jax
import jax.numpy as jnp
from jax import lax
from jax.experimental import pallas as pl
from jax.experimental.pallas import tpu as pltpu, tpu_sc as plsc

_L = 16


def _sc_body(x_hbm, out_hbm, vals_v, row_v, sem, *, img_w, img_h):
    cid = lax.axis_index("c")
    sid = lax.axis_index("s")

    @pl.when((cid == 0) & (sid == 0))
    def _work():
        cp = pltpu.make_async_copy(x_hbm, vals_v, sem)
        cp.start()
        cp.wait()
        lanes = lax.broadcasted_iota(jnp.int32, (_L,), 0)
        lane0 = lanes == 0
        zeros = jnp.zeros((_L,), jnp.float32)

        def lane0_scalar(c):
            return lax.reduce(
                jnp.where(lane0, vals_v[c], zeros), 0.0, lax.add, (0,)
            )

        cx = lane0_scalar(0)
        cy = lane0_scalar(1)
        dw = lane0_scalar(2) * 0.5
        dh = lane0_scalar(3) * 0.5
        score = lane0_scalar(4)
        x1 = jnp.clip(cx - dw, 0.0, img_w)
        y1 = jnp.clip(cy - dh, 0.0, img_h)
        x2 = jnp.clip(cx + dw, 0.0, img_w)
        y2 = jnp.clip(cy + dh, 0.0, img_h)
        row = zeros
        for k, v in enumerate((x1, y1, x2, y2, score)):
            row = jnp.where(lanes == k + 1, v, row)
        row_v[...] = row
        pltpu.make_async_copy(row_v.at[pl.ds(0, 7)], out_hbm.at[0], sem).start()
        pltpu.make_async_copy(row_v.at[pl.ds(0, 7)], out_hbm.at[0], sem).wait()


def kernel(x):
    img_h, img_w = float(x.shape[2]), float(x.shape[3])
    tile = jax.lax.slice(x, (0, 0, 0, 0), (1, 5, 1, _L)).reshape(5, _L)
    body = functools.partial(_sc_body, img_w=img_w, img_h=img_h)
    mesh = plsc.VectorSubcoreMesh(core_axis_name="c", subcore_axis_name="s")
    return pl.kernel(
        body,
        mesh=mesh,
        compiler_params=pltpu.CompilerParams(needs_layout_passes=False),
        out_type=jax.ShapeDtypeStruct((1, 7), jnp.float32),
        scratch_types=[
            pltpu.VMEM((5, _L), jnp.float32),
            pltpu.VMEM((_L,), jnp.float32),
            pltpu.SemaphoreType.DMA,
        ],
    )(tile)


# speedup vs baseline: 21.7003x; 7.2734x over previous
import functools

import jax
import jax.numpy as jnp
from jax.experimental import pallas as pl


def _decode_kernel(x_ref, o_ref, *, img_w, img_h):
    cx = x_ref[0, 0]
    cy = x_ref[1, 0]
    dw = x_ref[2, 0] * 0.5
    dh = x_ref[3, 0] * 0.5
    score = x_ref[4, 0]
    x1 = jnp.clip(cx - dw, 0.0, img_w)
    y1 = jnp.clip(cy - dh, 0.0, img_h)
    x2 = jnp.clip(cx + dw, 0.0, img_w)
    y2 = jnp.clip(cy + dh, 0.0, img_h)
    col = jax.lax.broadcasted_iota(jnp.int32, (1, 7), 1)
    out = jnp.zeros((1, 7), jnp.float32)
    for i, v in enumerate((x1, y1, x2, y2, score)):
        out = jnp.where(col == i + 1, v, out)
    o_ref[...] = out


def kernel(x):
    img_h, img_w = float(x.shape[2]), float(x.shape[3])
    tile = jax.lax.slice(x, (0, 0, 0, 0), (1, 8, 1, 128)).reshape(8, 128)
    body = functools.partial(_decode_kernel, img_w=img_w, img_h=img_h)
    return pl.pallas_call(
        body,
        grid=(1,),
        in_specs=[pl.BlockSpec((8, 128), lambda i: (0, 0))],
        out_specs=pl.BlockSpec((1, 7), lambda i: (0, 0)),
        out_shape=jax.ShapeDtypeStruct((1, 7), jnp.float32),
    )(tile)


# speedup vs baseline: 22.0889x; 1.0179x over previous
import functools

import jax
import jax.numpy as jnp
from jax.experimental import pallas as pl


def _decode_kernel(x_ref, o_ref, *, img_w, img_h):
    cx = x_ref[0, 0]
    cy = x_ref[1, 0]
    dw = x_ref[2, 0] * 0.5
    dh = x_ref[3, 0] * 0.5
    score = x_ref[4, 0]
    x1 = jnp.clip(cx - dw, 0.0, img_w)
    y1 = jnp.clip(cy - dh, 0.0, img_h)
    x2 = jnp.clip(cx + dw, 0.0, img_w)
    y2 = jnp.clip(cy + dh, 0.0, img_h)
    col = jax.lax.broadcasted_iota(jnp.int32, (1, 7), 1)
    out = jnp.zeros((1, 7), jnp.float32)
    for i, v in enumerate((x1, y1, x2, y2, score)):
        out = jnp.where(col == i + 1, v, out)
    o_ref[...] = out


def kernel(x):
    img_h, img_w = float(x.shape[2]), float(x.shape[3])
    tile = jax.lax.slice(x, (0, 0, 0, 0), (1, 5, 1, 128)).reshape(5, 128)
    body = functools.partial(_decode_kernel, img_w=img_w, img_h=img_h)
    return pl.pallas_call(
        body,
        grid=(1,),
        in_specs=[pl.BlockSpec((5, 128), lambda i: (0, 0))],
        out_specs=pl.BlockSpec((1, 7), lambda i: (0, 0)),
        out_shape=jax.ShapeDtypeStruct((1, 7), jnp.float32),
    )(tile)
